# Initial kernel scaffold; baseline (speedup 1.0000x reference)
import jax
import jax.numpy as jnp
from jax.experimental import pallas as pl


def kernel(x, adj, W1, b1, W2, b2):
    raise NotImplementedError("write your pallas kernel here")


# speedup vs baseline: 5.6291x; 5.6291x over previous
import functools

import jax
import jax.numpy as jnp
from jax import lax
from jax.experimental import pallas as pl
from jax.experimental.pallas import tpu as pltpu
from jax.experimental.pallas import tpu_sc as plsc

N = 10000
E = 160000
D = 256
DH = 128
NROWS = 10240
ROWS_PER_TILE = NROWS // 16
CHUNK = 128
NCHUNK_SEG = 80
NCHUNK_DEG = 40
DEGW = 16


def _zero_rows(buf, nrows, width):
    z = jnp.zeros((16,), jnp.float32)

    def body(r, carry):
        for k in range(width // 16):
            buf[r, pl.ds(k * 16, 16)] = z
        return carry

    lax.fori_loop(0, nrows, body, 0)


def _deg_body(dst_hbm, ridx_hbm, out_hbm, dst_v, ridx_v, ones_v, zbuf, rbuf,
              acc, sem):
    c = lax.axis_index("c")
    s = lax.axis_index("s")
    pltpu.sync_copy(dst_hbm.at[c, s], dst_v)
    pltpu.sync_copy(ridx_hbm.at[s], ridx_v)
    oh = jnp.where(lax.iota(jnp.int32, 16) == 0, 1.0, 0.0)

    def mkones(r, carry):
        ones_v[r] = oh
        return carry

    lax.fori_loop(0, CHUNK, mkones, 0)
    _zero_rows(zbuf, CHUNK, DEGW)
    for k in range(ROWS_PER_TILE // CHUNK):
        pltpu.sync_copy(zbuf, acc.at[ridx_v.at[k]])
    plsc.subcore_barrier()

    def chunk(q, carry):
        pltpu.sync_copy(ones_v, acc.at[dst_v.at[q]], add=True)
        return carry

    lax.fori_loop(0, NCHUNK_DEG, chunk, 0)
    plsc.subcore_barrier()
    for k in range(ROWS_PER_TILE // CHUNK):
        pltpu.async_copy(acc.at[ridx_v.at[k]], rbuf, sem).wait()
        row0 = s * ROWS_PER_TILE + k * CHUNK
        pltpu.sync_copy(rbuf, out_hbm.at[c, pl.ds(row0, CHUNK)])


def _seg_body(g_hbm, src_hbm, dst_hbm, ridx_hbm, out_hbm,
              src_v, dst_v, ridx_v, buf, acc, sem):
    c = lax.axis_index("c")
    s = lax.axis_index("s")
    pltpu.sync_copy(src_hbm.at[c, s], src_v)
    pltpu.sync_copy(dst_hbm.at[s], dst_v)
    pltpu.sync_copy(ridx_hbm.at[s], ridx_v)
    _zero_rows(buf, CHUNK, DH)
    for k in range(ROWS_PER_TILE // CHUNK):
        pltpu.sync_copy(buf, acc.at[ridx_v.at[k]])
    plsc.subcore_barrier()

    def chunk(q, carry):
        pltpu.async_copy(g_hbm.at[src_v.at[q]], buf, sem).wait()
        pltpu.sync_copy(buf, acc.at[dst_v.at[q]], add=True)
        return carry

    lax.fori_loop(0, NCHUNK_SEG, chunk, 0)
    plsc.subcore_barrier()
    for k in range(ROWS_PER_TILE // CHUNK):
        pltpu.async_copy(acc.at[ridx_v.at[k]], buf, sem).wait()
        row0 = s * ROWS_PER_TILE + k * CHUNK
        pltpu.sync_copy(buf, out_hbm.at[c, pl.ds(row0, CHUNK)])


@functools.cache
def _mesh():
    return plsc.VectorSubcoreMesh(core_axis_name="c", subcore_axis_name="s")


@functools.cache
def _deg_call():
    return pl.kernel(
        _deg_body,
        out_type=jax.ShapeDtypeStruct((2, NROWS, DEGW), jnp.float32),
        mesh=_mesh(),
        scratch_types=[
            pltpu.VMEM((NCHUNK_DEG, CHUNK), jnp.int32),
            pltpu.VMEM((ROWS_PER_TILE // CHUNK, CHUNK), jnp.int32),
            pltpu.VMEM((CHUNK, DEGW), jnp.float32),
            pltpu.VMEM((CHUNK, DEGW), jnp.float32),
            pltpu.VMEM((CHUNK, DEGW), jnp.float32),
            pltpu.VMEM_SHARED((NROWS, DEGW), jnp.float32),
            pltpu.SemaphoreType.DMA,
        ],
    )


@functools.cache
def _seg_call():
    return pl.kernel(
        _seg_body,
        out_type=jax.ShapeDtypeStruct((2, NROWS, DH), jnp.float32),
        mesh=_mesh(),
        scratch_types=[
            pltpu.VMEM((NCHUNK_SEG, CHUNK), jnp.int32),
            pltpu.VMEM((NCHUNK_SEG, CHUNK), jnp.int32),
            pltpu.VMEM((ROWS_PER_TILE // CHUNK, CHUNK), jnp.int32),
            pltpu.VMEM((CHUNK, DH), jnp.float32),
            pltpu.VMEM_SHARED((NROWS, DH), jnp.float32),
            pltpu.SemaphoreType.DMA,
        ],
    )


def _dinv_of(dega):
    deg = jnp.sum(dega[0], axis=1) + jnp.sum(dega[1], axis=1) + 1.0
    return lax.rsqrt(deg)


def _mm1_body(x_ref, w_ref, dega_ref, g_ref):
    h = jnp.dot(x_ref[...], w_ref[...], preferred_element_type=jnp.float32)
    dinv = _dinv_of(dega_ref[...])
    g_ref[0] = h * dinv[:, None]


def _mid_body(s_ref, g_ref, dega_ref, b1_ref, w2_ref, g2_ref):
    dinv = _dinv_of(dega_ref[...])
    b1 = b1_ref[...]
    t0 = dinv[:, None] * (s_ref[0] + g_ref[0]) + b1[0, :DH]
    t1 = dinv[:, None] * (s_ref[1] + g_ref[1]) + b1[0, DH:]
    r = jnp.maximum(jnp.concatenate([t0, t1], axis=1), 0.0)
    h2 = jnp.dot(r, w2_ref[...], preferred_element_type=jnp.float32)
    g2_ref[0] = h2[:, :DH] * dinv[:, None]
    g2_ref[1] = h2[:, DH:] * dinv[:, None]


def _out_body(s_ref, g_ref, dega_ref, b2_ref, out_ref):
    dinv = _dinv_of(dega_ref[...])
    b2 = b2_ref[...]
    o0 = dinv[:, None] * (s_ref[0] + g_ref[0]) + b2[0, :DH]
    o1 = dinv[:, None] * (s_ref[1] + g_ref[1]) + b2[0, DH:]
    out_ref[...] = jnp.concatenate([o0, o1], axis=1)


_RB = 1000
_GRID = N // _RB


def _mm1(x, w1, dega):
    return pl.pallas_call(
        _mm1_body,
        grid=(_GRID, 2),
        in_specs=[
            pl.BlockSpec((_RB, D), lambda i, j: (i, 0)),
            pl.BlockSpec((D, DH), lambda i, j: (0, j)),
            pl.BlockSpec((2, _RB, DEGW), lambda i, j: (0, i, 0)),
        ],
        out_specs=pl.BlockSpec((1, _RB, DH), lambda i, j: (j, i, 0)),
        out_shape=jax.ShapeDtypeStruct((2, N, DH), jnp.float32),
    )(x, w1, dega)


def _mid(s1, g1, dega, b1, w2):
    return pl.pallas_call(
        _mid_body,
        grid=(_GRID,),
        in_specs=[
            pl.BlockSpec((2, _RB, DH), lambda i: (0, i, 0)),
            pl.BlockSpec((2, _RB, DH), lambda i: (0, i, 0)),
            pl.BlockSpec((2, _RB, DEGW), lambda i: (0, i, 0)),
            pl.BlockSpec((1, D), lambda i: (0, 0)),
            pl.BlockSpec((D, D), lambda i: (0, 0)),
        ],
        out_specs=pl.BlockSpec((2, _RB, DH), lambda i: (0, i, 0)),
        out_shape=jax.ShapeDtypeStruct((2, N, DH), jnp.float32),
    )(s1, g1, dega, b1, w2)


def _outk(s2, g2, dega, b2):
    return pl.pallas_call(
        _out_body,
        grid=(_GRID,),
        in_specs=[
            pl.BlockSpec((2, _RB, DH), lambda i: (0, i, 0)),
            pl.BlockSpec((2, _RB, DH), lambda i: (0, i, 0)),
            pl.BlockSpec((2, _RB, DEGW), lambda i: (0, i, 0)),
            pl.BlockSpec((1, D), lambda i: (0, 0)),
        ],
        out_specs=pl.BlockSpec((_RB, D), lambda i: (i, 0)),
        out_shape=jax.ShapeDtypeStruct((N, D), jnp.float32),
    )(s2, g2, dega, b2)


@jax.jit
def kernel(x, adj, W1, b1, W2, b2):
    src = adj[0]
    dst = adj[1]
    pad = NCHUNK_SEG * CHUNK * 16 - E
    srcp = jnp.pad(src, (0, pad)).reshape(16, NCHUNK_SEG, CHUNK)
    src_seg = jnp.stack([srcp, srcp + N])
    dstp = jnp.pad(dst, (0, pad), constant_values=N)
    dst_seg = dstp.reshape(16, NCHUNK_SEG, CHUNK)
    dst_deg = dstp.reshape(2, 16, NCHUNK_DEG, CHUNK)

    rowidx = jnp.arange(NROWS, dtype=jnp.int32).reshape(16, 5, CHUNK)
    dega = _deg_call()(dst_deg, rowidx)
    g1 = _mm1(x, W1, dega)
    s1 = _seg_call()(g1.reshape(2 * N, DH), src_seg, dst_seg, rowidx)
    g2 = _mid(s1, g1, dega, b1.reshape(1, D), W2)
    s2 = _seg_call()(g2.reshape(2 * N, DH), src_seg, dst_seg, rowidx)
    return _outk(s2, g2, dega, b2.reshape(1, D))


# speedup vs baseline: 6.6588x; 1.1829x over previous
import functools

import jax
import jax.numpy as jnp
from jax import lax
from jax.experimental import pallas as pl
from jax.experimental.pallas import tpu as pltpu
from jax.experimental.pallas import tpu_sc as plsc

N = 10000
E = 160000
D = 256
DH = 128
NROWS = 10240
ROWS_PER_TILE = NROWS // 16
CHUNK = 128
NCHUNK_SEG = 80
NCHUNK_DEG = 40
DEGW = 16
NBUF = 2
WCHUNK = 40


def _zero_rows(buf, nrows, width):
    z = jnp.zeros((16,), jnp.float32)

    def body(r, carry):
        for k in range(width // 16):
            buf[r, pl.ds(k * 16, 16)] = z
        return carry

    lax.fori_loop(0, nrows, body, 0)


def _deg_body(dst_hbm, ridx_hbm, out_hbm, dst_v, ridx_v, ones_v, zbuf, rbuf,
              acc, sem):
    c = lax.axis_index("c")
    s = lax.axis_index("s")
    pltpu.sync_copy(dst_hbm.at[c, s], dst_v)
    pltpu.sync_copy(ridx_hbm.at[s], ridx_v)
    oh = jnp.where(lax.iota(jnp.int32, 16) == 0, 1.0, 0.0)

    def mkones(r, carry):
        ones_v[r] = oh
        return carry

    lax.fori_loop(0, CHUNK, mkones, 0)
    _zero_rows(zbuf, CHUNK, DEGW)
    for k in range(ROWS_PER_TILE // CHUNK):
        pltpu.sync_copy(zbuf, acc.at[ridx_v.at[k]])
    plsc.subcore_barrier()

    def chunk(q, carry):
        pltpu.sync_copy(ones_v, acc.at[dst_v.at[q]], add=True)
        return carry

    lax.fori_loop(0, NCHUNK_DEG, chunk, 0)
    plsc.subcore_barrier()
    for k in range(ROWS_PER_TILE // CHUNK):
        pltpu.async_copy(acc.at[ridx_v.at[k]], rbuf, sem).wait()
        row0 = s * ROWS_PER_TILE + k * CHUNK
        pltpu.sync_copy(rbuf, out_hbm.at[c, pl.ds(row0, CHUNK)])


def _seg_body(g_hbm, src_hbm, dst_hbm, ridx_hbm, out_hbm,
              src_v, dst_v, ridx_v, buf, acc, sem):
    c = lax.axis_index("c")
    s = lax.axis_index("s")
    pltpu.sync_copy(ridx_hbm.at[s], ridx_v)
    z = jnp.zeros((16,), jnp.float32)

    def zrow(r, carry):
        for k in range(DH // 16):
            buf[0, r, pl.ds(k * 16, 16)] = z
        return carry

    lax.fori_loop(0, CHUNK, zrow, 0)
    for k in range(ROWS_PER_TILE // CHUNK):
        pltpu.sync_copy(buf.at[0], acc.at[ridx_v.at[k]])
    plsc.subcore_barrier()

    for w in range(NCHUNK_SEG // WCHUNK):
        pltpu.sync_copy(src_hbm.at[c, s, pl.ds(w * WCHUNK, WCHUNK)], src_v)
        pltpu.sync_copy(dst_hbm.at[s, pl.ds(w * WCHUNK, WCHUNK)], dst_v)
        pltpu.async_copy(g_hbm.at[src_v.at[0]], buf.at[0], sem)

        def chunk_group(i, carry):
            for b in range(2):
                q = i * 2 + b

                @pl.when(q + 1 < WCHUNK)
                def _():
                    pltpu.async_copy(
                        g_hbm.at[src_v.at[q + 1]], buf.at[(b + 1) % 2], sem)

                pltpu.make_async_copy(
                    g_hbm.at[src_v.at[q]], buf.at[b], sem).wait()
                pltpu.sync_copy(buf.at[b], acc.at[dst_v.at[q]], add=True)
            return carry

        lax.fori_loop(0, WCHUNK // 2, chunk_group, 0)
    plsc.subcore_barrier()
    for k in range(ROWS_PER_TILE // CHUNK):
        pltpu.async_copy(acc.at[ridx_v.at[k]], buf.at[0], sem).wait()
        row0 = s * ROWS_PER_TILE + k * CHUNK
        pltpu.sync_copy(buf.at[0], out_hbm.at[c, pl.ds(row0, CHUNK)])


@functools.cache
def _mesh():
    return plsc.VectorSubcoreMesh(core_axis_name="c", subcore_axis_name="s")


@functools.cache
def _deg_call():
    return pl.kernel(
        _deg_body,
        out_type=jax.ShapeDtypeStruct((2, NROWS, DEGW), jnp.float32),
        mesh=_mesh(),
        scratch_types=[
            pltpu.VMEM((NCHUNK_DEG, CHUNK), jnp.int32),
            pltpu.VMEM((ROWS_PER_TILE // CHUNK, CHUNK), jnp.int32),
            pltpu.VMEM((CHUNK, DEGW), jnp.float32),
            pltpu.VMEM((CHUNK, DEGW), jnp.float32),
            pltpu.VMEM((CHUNK, DEGW), jnp.float32),
            pltpu.VMEM_SHARED((NROWS, DEGW), jnp.float32),
            pltpu.SemaphoreType.DMA,
        ],
    )


@functools.cache
def _seg_call():
    return pl.kernel(
        _seg_body,
        out_type=jax.ShapeDtypeStruct((2, NROWS, DH), jnp.float32),
        mesh=_mesh(),
        scratch_types=[
            pltpu.VMEM((WCHUNK, CHUNK), jnp.int32),
            pltpu.VMEM((WCHUNK, CHUNK), jnp.int32),
            pltpu.VMEM((ROWS_PER_TILE // CHUNK, CHUNK), jnp.int32),
            pltpu.VMEM((NBUF, CHUNK, DH), jnp.float32),
            pltpu.VMEM_SHARED((NROWS, DH), jnp.float32),
            pltpu.SemaphoreType.DMA,
        ],
    )


def _dinv_of(dega):
    deg = jnp.sum(dega[0], axis=1) + jnp.sum(dega[1], axis=1) + 1.0
    return lax.rsqrt(deg)


def _mm1_body(x_ref, w_ref, dega_ref, g_ref):
    h = jnp.dot(x_ref[...], w_ref[...], preferred_element_type=jnp.float32)
    dinv = _dinv_of(dega_ref[...])
    g_ref[0] = h * dinv[:, None]


def _mid_body(s_ref, g_ref, dega_ref, b1_ref, w2_ref, g2_ref):
    dinv = _dinv_of(dega_ref[...])
    b1 = b1_ref[...]
    t0 = dinv[:, None] * (s_ref[0] + g_ref[0]) + b1[0, :DH]
    t1 = dinv[:, None] * (s_ref[1] + g_ref[1]) + b1[0, DH:]
    r = jnp.maximum(jnp.concatenate([t0, t1], axis=1), 0.0)
    h2 = jnp.dot(r, w2_ref[...], preferred_element_type=jnp.float32)
    g2_ref[0] = h2[:, :DH] * dinv[:, None]
    g2_ref[1] = h2[:, DH:] * dinv[:, None]


def _out_body(s_ref, g_ref, dega_ref, b2_ref, out_ref):
    dinv = _dinv_of(dega_ref[...])
    b2 = b2_ref[...]
    o0 = dinv[:, None] * (s_ref[0] + g_ref[0]) + b2[0, :DH]
    o1 = dinv[:, None] * (s_ref[1] + g_ref[1]) + b2[0, DH:]
    out_ref[...] = jnp.concatenate([o0, o1], axis=1)


_RB = 1000
_GRID = N // _RB


def _mm1(x, w1, dega):
    return pl.pallas_call(
        _mm1_body,
        grid=(_GRID, 2),
        in_specs=[
            pl.BlockSpec((_RB, D), lambda i, j: (i, 0)),
            pl.BlockSpec((D, DH), lambda i, j: (0, j)),
            pl.BlockSpec((2, _RB, DEGW), lambda i, j: (0, i, 0)),
        ],
        out_specs=pl.BlockSpec((1, _RB, DH), lambda i, j: (j, i, 0)),
        out_shape=jax.ShapeDtypeStruct((2, N, DH), jnp.float32),
    )(x, w1, dega)


def _mid(s1, g1, dega, b1, w2):
    return pl.pallas_call(
        _mid_body,
        grid=(_GRID,),
        in_specs=[
            pl.BlockSpec((2, _RB, DH), lambda i: (0, i, 0)),
            pl.BlockSpec((2, _RB, DH), lambda i: (0, i, 0)),
            pl.BlockSpec((2, _RB, DEGW), lambda i: (0, i, 0)),
            pl.BlockSpec((1, D), lambda i: (0, 0)),
            pl.BlockSpec((D, D), lambda i: (0, 0)),
        ],
        out_specs=pl.BlockSpec((2, _RB, DH), lambda i: (0, i, 0)),
        out_shape=jax.ShapeDtypeStruct((2, N, DH), jnp.float32),
    )(s1, g1, dega, b1, w2)


def _outk(s2, g2, dega, b2):
    return pl.pallas_call(
        _out_body,
        grid=(_GRID,),
        in_specs=[
            pl.BlockSpec((2, _RB, DH), lambda i: (0, i, 0)),
            pl.BlockSpec((2, _RB, DH), lambda i: (0, i, 0)),
            pl.BlockSpec((2, _RB, DEGW), lambda i: (0, i, 0)),
            pl.BlockSpec((1, D), lambda i: (0, 0)),
        ],
        out_specs=pl.BlockSpec((_RB, D), lambda i: (i, 0)),
        out_shape=jax.ShapeDtypeStruct((N, D), jnp.float32),
    )(s2, g2, dega, b2)


@jax.jit
def kernel(x, adj, W1, b1, W2, b2):
    src = adj[0]
    dst = adj[1]
    pad = NCHUNK_SEG * CHUNK * 16 - E
    srcp = jnp.pad(src, (0, pad)).reshape(16, NCHUNK_SEG, CHUNK)
    src_seg = jnp.stack([srcp, srcp + N])
    dstp = jnp.pad(dst, (0, pad), constant_values=N)
    dst_seg = dstp.reshape(16, NCHUNK_SEG, CHUNK)
    dst_deg = dstp.reshape(2, 16, NCHUNK_DEG, CHUNK)

    rowidx = jnp.arange(NROWS, dtype=jnp.int32).reshape(16, 5, CHUNK)
    dega = _deg_call()(dst_deg, rowidx)
    g1 = _mm1(x, W1, dega)
    s1 = _seg_call()(g1.reshape(2 * N, DH), src_seg, dst_seg, rowidx)
    g2 = _mid(s1, g1, dega, b1.reshape(1, D), W2)
    s2 = _seg_call()(g2.reshape(2 * N, DH), src_seg, dst_seg, rowidx)
    return _outk(s2, g2, dega, b2.reshape(1, D))
